# baseline (speedup 1.0000x reference)
import functools

import jax
import jax.numpy as jnp
from jax import lax
from jax.experimental import pallas as pl
from jax.experimental.pallas import tpu as pltpu
from jax.experimental.pallas import tpu_sc as plsc

DEG = 32
D = 128
BLK = 400
C = 4
NW = 32


def _sc_reduce(neigh):
    n_nodes = neigh.shape[0]
    nc = n_nodes // C
    slots = -(-nc // NW)
    slots += slots % 2
    mesh = plsc.VectorSubcoreMesh(core_axis_name="c", subcore_axis_name="s")

    @functools.partial(
        pl.kernel,
        out_type=jax.ShapeDtypeStruct((n_nodes, D), jnp.float32),
        mesh=mesh,
        scratch_types=[
            pltpu.VMEM((C, DEG, D), jnp.float32),
            pltpu.VMEM((C, DEG, D), jnp.float32),
            pltpu.VMEM((C, D), jnp.float32),
            pltpu.VMEM((C, D), jnp.float32),
            pltpu.SemaphoreType.DMA,
            pltpu.SemaphoreType.DMA,
        ],
    )
    def reduce_kernel(neigh_hbm, aggr_hbm, buf0, buf1, out0, out1, sem0, sem1):
        wid = lax.axis_index("s") * 2 + lax.axis_index("c")

        def chunk_of(i):
            return lax.rem(i * NW + wid, nc)

        def start(i, buf, sem):
            c = chunk_of(i)
            pltpu.make_async_copy(
                neigh_hbm.at[pl.ds(c * C, C)], buf, sem).start()

        def wait(i, buf, sem):
            c = chunk_of(i)
            pltpu.make_async_copy(
                neigh_hbm.at[pl.ds(c * C, C)], buf, sem).wait()

        def compute_store(i, buf, out):
            for n in range(C):
                for j in range(D // 16):
                    sl = pl.ds(j * 16, 16)
                    acc = buf[n, 0, sl]
                    for k in range(1, DEG):
                        acc = acc + buf[n, k, sl]
                    out[n, sl] = acc
            c = chunk_of(i)
            pltpu.sync_copy(out, aggr_hbm.at[pl.ds(c * C, C)])

        start(0, buf0, sem0)
        start(1, buf1, sem1)

        def body(p, _):
            i0 = 2 * p
            i1 = 2 * p + 1
            wait(i0, buf0, sem0)
            compute_store(i0, buf0, out0)

            @pl.when(i0 + 2 < slots)
            def _():
                start(i0 + 2, buf0, sem0)

            wait(i1, buf1, sem1)
            compute_store(i1, buf1, out1)

            @pl.when(i1 + 2 < slots)
            def _():
                start(i1 + 2, buf1, sem1)

        lax.fori_loop(0, slots // 2, body, None)

    return reduce_kernel(neigh)


def _tc_combine_body(src_ref, aggr_ref, w_ref, b_ref, out_ref):
    h = jnp.dot(aggr_ref[...] * (1.0 / DEG), w_ref[...],
                preferred_element_type=jnp.float32)
    h = h + jnp.dot(src_ref[...], b_ref[...],
                    preferred_element_type=jnp.float32)
    out_ref[...] = jnp.maximum(h, 0.0)


def _tc_combine(src, aggr_sum, W_agg, b):
    n = src.shape[0]
    return pl.pallas_call(
        _tc_combine_body,
        grid=(n // BLK,),
        in_specs=[
            pl.BlockSpec((BLK, D), lambda i: (i, 0)),
            pl.BlockSpec((BLK, D), lambda i: (i, 0)),
            pl.BlockSpec((D, D), lambda i: (0, 0)),
            pl.BlockSpec((D, D), lambda i: (0, 0)),
        ],
        out_specs=pl.BlockSpec((BLK, D), lambda i: (i, 0)),
        out_shape=jax.ShapeDtypeStruct((n, D), jnp.float32),
    )(src, aggr_sum, W_agg, b)


def kernel(src_node_features, neighbor_node_features, W_agg, b):
    aggr_sum = _sc_reduce(neighbor_node_features)
    return _tc_combine(src_node_features, aggr_sum, W_agg, b)


# speedup vs baseline: 2.9079x; 2.9079x over previous
import functools

import jax
import jax.numpy as jnp
from jax import lax
from jax.experimental import pallas as pl
from jax.experimental.pallas import tpu as pltpu
from jax.experimental.pallas import tpu_sc as plsc

DEG = 32
D = 128
BLK = 400
C = 4
NW = 32
SC_SHARE = 2000


def _sc_reduce(neigh, n_out):
    nc = n_out // C
    slots = -(-nc // NW)
    slots += slots % 2
    mesh = plsc.VectorSubcoreMesh(core_axis_name="c", subcore_axis_name="s")

    @functools.partial(
        pl.kernel,
        out_type=jax.ShapeDtypeStruct((n_out, D), jnp.float32),
        mesh=mesh,
        scratch_types=[
            pltpu.VMEM((C, DEG, D), jnp.float32),
            pltpu.VMEM((C, DEG, D), jnp.float32),
            pltpu.VMEM((C, D), jnp.float32),
            pltpu.VMEM((C, D), jnp.float32),
            pltpu.SemaphoreType.DMA,
            pltpu.SemaphoreType.DMA,
        ],
    )
    def reduce_kernel(neigh_hbm, aggr_hbm, buf0, buf1, out0, out1, sem0, sem1):
        wid = lax.axis_index("s") * 2 + lax.axis_index("c")

        def chunk_of(i):
            return lax.rem(i * NW + wid, nc)

        def start(i, buf, sem):
            c = chunk_of(i)
            pltpu.make_async_copy(
                neigh_hbm.at[pl.ds(c * C, C)], buf, sem).start()

        def wait(i, buf, sem):
            c = chunk_of(i)
            pltpu.make_async_copy(
                neigh_hbm.at[pl.ds(c * C, C)], buf, sem).wait()

        def compute_store(i, buf, out):
            for n in range(C):
                for j in range(D // 16):
                    sl = pl.ds(j * 16, 16)
                    acc = buf[n, 0, sl]
                    for k in range(1, DEG):
                        acc = acc + buf[n, k, sl]
                    out[n, sl] = acc
            c = chunk_of(i)
            pltpu.sync_copy(out, aggr_hbm.at[pl.ds(c * C, C)])

        start(0, buf0, sem0)
        start(1, buf1, sem1)

        def body(p, _):
            i0 = 2 * p
            i1 = 2 * p + 1
            wait(i0, buf0, sem0)
            compute_store(i0, buf0, out0)

            @pl.when(i0 + 2 < slots)
            def _():
                start(i0 + 2, buf0, sem0)

            wait(i1, buf1, sem1)
            compute_store(i1, buf1, out1)

            @pl.when(i1 + 2 < slots)
            def _():
                start(i1 + 2, buf1, sem1)

        lax.fori_loop(0, slots // 2, body, None)

    return reduce_kernel(neigh)


def _tc_fused_body(src_ref, neigh_ref, w_ref, b_ref, out_ref):
    aggr = jnp.sum(neigh_ref[...], axis=1) * (1.0 / DEG)
    h = jnp.dot(aggr, w_ref[...], preferred_element_type=jnp.float32)
    h = h + jnp.dot(src_ref[...], b_ref[...], preferred_element_type=jnp.float32)
    out_ref[...] = jnp.maximum(h, 0.0)


def _tc_fused(src, neigh, W_agg, b, start_node):
    n = src.shape[0]
    off = start_node // BLK
    return pl.pallas_call(
        _tc_fused_body,
        grid=((n - start_node) // BLK,),
        in_specs=[
            pl.BlockSpec((BLK, D), lambda i: (i + off, 0)),
            pl.BlockSpec((BLK, DEG, D), lambda i: (i + off, 0, 0)),
            pl.BlockSpec((D, D), lambda i: (0, 0)),
            pl.BlockSpec((D, D), lambda i: (0, 0)),
        ],
        out_specs=pl.BlockSpec((BLK, D), lambda i: (i, 0)),
        out_shape=jax.ShapeDtypeStruct((n - start_node, D), jnp.float32),
    )(src, neigh, W_agg, b)


def _tc_combine_body(src_ref, aggr_ref, w_ref, b_ref, out_ref):
    h = jnp.dot(aggr_ref[...] * (1.0 / DEG), w_ref[...],
                preferred_element_type=jnp.float32)
    h = h + jnp.dot(src_ref[...], b_ref[...],
                    preferred_element_type=jnp.float32)
    out_ref[...] = jnp.maximum(h, 0.0)


def _tc_combine(src, aggr_sum, W_agg, b):
    s = aggr_sum.shape[0]
    return pl.pallas_call(
        _tc_combine_body,
        grid=(s // BLK,),
        in_specs=[
            pl.BlockSpec((BLK, D), lambda i: (i, 0)),
            pl.BlockSpec((BLK, D), lambda i: (i, 0)),
            pl.BlockSpec((D, D), lambda i: (0, 0)),
            pl.BlockSpec((D, D), lambda i: (0, 0)),
        ],
        out_specs=pl.BlockSpec((BLK, D), lambda i: (i, 0)),
        out_shape=jax.ShapeDtypeStruct((s, D), jnp.float32),
    )(src, aggr_sum, W_agg, b)


def kernel(src_node_features, neighbor_node_features, W_agg, b):
    aggr_top = _sc_reduce(neighbor_node_features, SC_SHARE)
    out_bot = _tc_fused(src_node_features, neighbor_node_features,
                        W_agg, b, SC_SHARE)
    out_top = _tc_combine(src_node_features, aggr_top, W_agg, b)
    return jnp.concatenate([out_top, out_bot], axis=0)


# speedup vs baseline: 4.5984x; 1.5813x over previous
import jax
import jax.numpy as jnp
from jax.experimental import pallas as pl

DEG = 32
D = 128
BLK = 400


def _body(src_ref, neigh_ref, w_ref, b_ref, out_ref):
    aggr = jnp.sum(neigh_ref[...], axis=1) * (1.0 / DEG)
    h = jnp.dot(aggr, w_ref[...], preferred_element_type=jnp.float32)
    h = h + jnp.dot(src_ref[...], b_ref[...], preferred_element_type=jnp.float32)
    out_ref[...] = jnp.maximum(h, 0.0)


def kernel(src_node_features, neighbor_node_features, W_agg, b):
    n = src_node_features.shape[0]
    grid = (n // BLK,)
    return pl.pallas_call(
        _body,
        grid=grid,
        in_specs=[
            pl.BlockSpec((BLK, D), lambda i: (i, 0)),
            pl.BlockSpec((BLK, DEG, D), lambda i: (i, 0, 0)),
            pl.BlockSpec((D, D), lambda i: (0, 0)),
            pl.BlockSpec((D, D), lambda i: (0, 0)),
        ],
        out_specs=pl.BlockSpec((BLK, D), lambda i: (i, 0)),
        out_shape=jax.ShapeDtypeStruct((n, D), jnp.float32),
    )(src_node_features, neighbor_node_features, W_agg, b)
